# Initial kernel scaffold; baseline (speedup 1.0000x reference)
import jax
import jax.numpy as jnp
from jax.experimental import pallas as pl


def kernel(feat, coord, batch, W1, b1, gamma, beta, W2, b2):
    raise NotImplementedError("write your pallas kernel here")


# speedup vs baseline: 37.3189x; 37.3189x over previous
import functools

import jax
import jax.numpy as jnp
from jax import lax
from jax.experimental import pallas as pl
from jax.experimental.pallas import tpu as pltpu

_K = 32
_INF_BITS = 0x7F800000


def _geom_kernel(coord_row_ref, coord_t_ref, batch_row_ref, batch_col_ref,
                 lin_ref, den_ref, bits_scratch, *, block_rows):
    i = pl.program_id(0)
    n = coord_t_ref.shape[1]
    br = block_rows

    xr = coord_row_ref[:, 0:1]
    yr = coord_row_ref[:, 1:2]
    zr = coord_row_ref[:, 2:3]
    xc = coord_t_ref[0:1, :]
    yc = coord_t_ref[1:2, :]
    zc = coord_t_ref[2:3, :]

    bf = jnp.bfloat16
    xrq = xr.astype(bf).astype(jnp.float32)
    yrq = yr.astype(bf).astype(jnp.float32)
    zrq = zr.astype(bf).astype(jnp.float32)
    xcq = xc.astype(bf).astype(jnp.float32)
    ycq = yc.astype(bf).astype(jnp.float32)
    zcq = zc.astype(bf).astype(jnp.float32)
    sqr = xr * xr + yr * yr + zr * zr
    sqc = xc * xc + yc * yc + zc * zc
    dot = (xrq * xcq + yrq * ycq) + zrq * zcq
    d2 = jnp.maximum((sqr + sqc) - 2.0 * dot, 0.0)

    b_row = batch_row_ref[:, 0:1]
    b_col = batch_col_ref[0:1, :]
    same = b_row == b_col
    row_id = i * br + lax.broadcasted_iota(jnp.int32, (br, n), 0)
    col_id = lax.broadcasted_iota(jnp.int32, (br, n), 1)
    valid = same & (row_id != col_id)

    inf = jnp.float32(jnp.inf)
    d2m = jnp.where(valid, d2, inf)
    bits_scratch[:, :] = lax.bitcast_convert_type(d2m, jnp.int32)
    bits = bits_scratch[:, :]
    d2m = lax.bitcast_convert_type(bits, jnp.float32)

    cnt = jnp.sum(same.astype(jnp.int32), axis=1, keepdims=True)
    m = jnp.where(cnt > _K, _K, _K - 1)

    lo0 = jnp.full((br, 1), -1, jnp.int32)
    hi0 = jnp.full((br, 1), _INF_BITS, jnp.int32)

    def search_body(_, carry):
        lo, hi = carry
        mid = lo + ((hi - lo) >> 1)
        c = jnp.sum((bits <= mid).astype(jnp.int32), axis=1, keepdims=True)
        ge = c >= m
        return jnp.where(ge, lo, mid), jnp.where(ge, mid, hi)

    _, t = lax.fori_loop(0, 31, search_body, (lo0, hi0))

    maskb = bits <= t
    maskf = maskb.astype(jnp.float32)
    mf = m.astype(jnp.float32)
    inv_m = 1.0 / mf

    mx = jnp.sum(maskf * xc, axis=1, keepdims=True) * inv_m
    my = jnp.sum(maskf * yc, axis=1, keepdims=True) * inv_m
    mz = jnp.sum(maskf * zc, axis=1, keepdims=True) * inv_m

    cx = (xc - mx).astype(bf).astype(jnp.float32)
    cy = (yc - my).astype(bf).astype(jnp.float32)
    cz = (zc - mz).astype(bf).astype(jnp.float32)
    r_km1 = jnp.float32(1.0 / (_K - 1))
    cxx = jnp.sum(maskf * (cx * cx), axis=1, keepdims=True) * r_km1
    cyy = jnp.sum(maskf * (cy * cy), axis=1, keepdims=True) * r_km1
    czz = jnp.sum(maskf * (cz * cz), axis=1, keepdims=True) * r_km1
    cxy = jnp.sum(maskf * (cx * cy), axis=1, keepdims=True) * r_km1
    cxz = jnp.sum(maskf * (cx * cz), axis=1, keepdims=True) * r_km1
    cyz = jnp.sum(maskf * (cy * cz), axis=1, keepdims=True) * r_km1

    dist = jnp.where(d2m > 1e-12, jnp.sqrt(d2m), 0.0)
    sd = jnp.sum(jnp.where(maskb, dist, 0.0), axis=1, keepdims=True)

    tr = cxx + cyy + czz
    pm = (cxx * cyy - cxy * cxy) + (cxx * czz - cxz * cxz) + (cyy * czz - cyz * cyz)
    det = (cxx * (cyy * czz - cyz * cyz)
           - cxy * (cxy * czz - cyz * cxz)
           + cxz * (cxy * cyz - cyy * cxz))

    q = tr * (1.0 / 3.0)
    p2 = ((cxx - q) ** 2 + (cyy - q) ** 2 + (czz - q) ** 2
          + 2.0 * (cxy * cxy + cxz * cxz + cyz * cyz))
    p = jnp.sqrt(jnp.maximum(p2, 0.0) * (1.0 / 6.0))

    def newton_body(_, x):
        f = ((x - tr) * x + pm) * x - det
        fp = (3.0 * x - 2.0 * tr) * x + pm
        delta = jnp.where(jnp.abs(fp) > 1e-30, f / fp, 0.0)
        return jnp.maximum(x - delta, q)

    lam = lax.fori_loop(0, 12, newton_body, q + 2.0 * p)

    lin = (2.0 * lam - tr) / tr
    den = 1.0 / (sd * inv_m + 1e-6)

    ok = cnt >= _K
    lin_ref[:, :] = jnp.where(ok, lin, 0.0)
    den_ref[:, :] = jnp.where(ok, den, 0.0)


def _mlp_kernel(feat_ref, w1t_ref, b1_ref, gamma_ref, beta_ref,
                w2t_ref, b2_ref, lin_ref, den_ref, out_ref):
    bf = jnp.bfloat16
    h = jnp.dot(feat_ref[:, :].astype(bf), w1t_ref[:, :].astype(bf),
                preferred_element_type=jnp.float32) + b1_ref[0:1, :]
    mu = jnp.mean(h, axis=0, keepdims=True)
    var = jnp.mean((h - mu) ** 2, axis=0, keepdims=True)
    hn = (h - mu) / jnp.sqrt(var + 1e-5) * gamma_ref[0:1, :] + beta_ref[0:1, :]
    hn = jnp.maximum(hn, 0.0)
    logits = jnp.dot(hn.astype(bf), w2t_ref[:, :].astype(bf),
                     preferred_element_type=jnp.float32) + b2_ref[0:1, :]

    mx = jnp.max(logits, axis=1, keepdims=True)
    e = jnp.exp(logits - mx)
    probs = e / jnp.sum(e, axis=1, keepdims=True)
    p0 = probs[:, 0:1]
    p1 = probs[:, 1:2]
    p2 = probs[:, 2:3]

    lin = lin_ref[:, :]
    den = den_ref[:, :]
    third = jnp.float32(1.0 / 3.0)
    tower = (den * 2.0 + p0) * third
    background = (jnp.maximum(1.0 - lin, 1.0 - den) + p1) * third
    line = (lin * 2.0 + p2) * third

    c01 = tower * 0.1 + background * 0.5 + line * 0.2 + 1e-6
    c2 = tower * 0.1 + background * 0.5 + line * 5.0 + 1e-6
    out_ref[:, :] = jnp.concatenate([c01, c01, c2], axis=1)


def kernel(feat, coord, batch, W1, b1, gamma, beta, W2, b2):
    n, _ = feat.shape
    batch = batch.astype(jnp.int32)
    coord = coord.astype(jnp.float32)

    block_rows = 128
    grid = n // block_rows

    coord_t = coord.T
    batch_row = batch.reshape(n, 1)
    batch_col = batch.reshape(1, n)

    lin, den = pl.pallas_call(
        functools.partial(_geom_kernel, block_rows=block_rows),
        grid=(grid,),
        in_specs=[
            pl.BlockSpec((block_rows, 3), lambda i: (i, 0)),
            pl.BlockSpec((3, n), lambda i: (0, 0)),
            pl.BlockSpec((block_rows, 1), lambda i: (i, 0)),
            pl.BlockSpec((1, n), lambda i: (0, 0)),
        ],
        out_specs=[
            pl.BlockSpec((block_rows, 1), lambda i: (i, 0)),
            pl.BlockSpec((block_rows, 1), lambda i: (i, 0)),
        ],
        out_shape=[
            jax.ShapeDtypeStruct((n, 1), jnp.float32),
            jax.ShapeDtypeStruct((n, 1), jnp.float32),
        ],
        scratch_shapes=[pltpu.VMEM((block_rows, n), jnp.int32)],
    )(coord, coord_t, batch_row, batch_col)

    out = pl.pallas_call(
        _mlp_kernel,
        out_shape=jax.ShapeDtypeStruct((n, 3), jnp.float32),
    )(feat.astype(jnp.float32), W1.T.astype(jnp.float32),
      b1.reshape(1, -1).astype(jnp.float32),
      gamma.reshape(1, -1).astype(jnp.float32),
      beta.reshape(1, -1).astype(jnp.float32),
      W2.T.astype(jnp.float32), b2.reshape(1, -1).astype(jnp.float32),
      lin, den)
    return out


# speedup vs baseline: 52.3950x; 1.4040x over previous
import functools

import jax
import jax.numpy as jnp
from jax import lax
from jax.experimental import pallas as pl
from jax.experimental.pallas import tpu as pltpu

_K = 32
_INF_BITS = 0x7F800000
_CW = 1024


def _geom_kernel(win_ref, coord_row_ref, coord_t_ref, batch_row_ref,
                 batch_col_ref, lin_ref, den_ref, bits_scratch, *,
                 block_rows):
    i = pl.program_id(0)
    br = block_rows
    bf = jnp.bfloat16

    xr = coord_row_ref[:, 0:1]
    yr = coord_row_ref[:, 1:2]
    zr = coord_row_ref[:, 2:3]
    xrq = xr.astype(bf).astype(jnp.float32)
    yrq = yr.astype(bf).astype(jnp.float32)
    zrq = zr.astype(bf).astype(jnp.float32)
    sqr = xr * xr + yr * yr + zr * zr
    b_row = batch_row_ref[:, 0:1]
    row_id = i * br + lax.broadcasted_iota(jnp.int32, (br, _CW), 0)
    chunk_iota = lax.broadcasted_iota(jnp.int32, (br, _CW), 1)

    c0 = win_ref[2 * i]
    c1 = win_ref[2 * i + 1]

    def dist_body(ci, cnt):
        sl = pl.ds(ci * _CW, _CW)
        xc = coord_t_ref[0:1, sl]
        yc = coord_t_ref[1:2, sl]
        zc = coord_t_ref[2:3, sl]
        dot = (xrq * xc.astype(bf).astype(jnp.float32)
               + yrq * yc.astype(bf).astype(jnp.float32)) \
            + zrq * zc.astype(bf).astype(jnp.float32)
        sqc = xc * xc + yc * yc + zc * zc
        d2 = jnp.maximum((sqr + sqc) - 2.0 * dot, 0.0)
        same = b_row == batch_col_ref[0:1, sl]
        col_id = ci * _CW + chunk_iota
        valid = same & (row_id != col_id)
        d2m = jnp.where(valid, d2, jnp.float32(jnp.inf))
        bits_scratch[:, sl] = lax.bitcast_convert_type(d2m, jnp.int32)
        return cnt + jnp.sum(same.astype(jnp.int32), axis=1, keepdims=True)

    cnt = lax.fori_loop(c0, c1 + 1, dist_body,
                        jnp.zeros((br, 1), jnp.int32))
    m = jnp.where(cnt > _K, _K, _K - 1)

    lo0 = jnp.full((br, 1), -1, jnp.int32)
    hi0 = jnp.full((br, 1), _INF_BITS, jnp.int32)

    def count_le(mid):
        def body(ci, c):
            sl = pl.ds(ci * _CW, _CW)
            bits = bits_scratch[:, sl]
            return c + jnp.sum((bits <= mid).astype(jnp.int32), axis=1,
                               keepdims=True)
        return lax.fori_loop(c0, c1 + 1, body, jnp.zeros((br, 1), jnp.int32))

    def search_body(_, carry):
        lo, hi = carry
        mid = lo + ((hi - lo) >> 1)
        ge = count_le(mid) >= m
        return jnp.where(ge, lo, mid), jnp.where(ge, mid, hi)

    _, t = lax.fori_loop(0, 31, search_body, (lo0, hi0))

    mf = m.astype(jnp.float32)
    inv_m = 1.0 / mf

    def mean_body(ci, carry):
        sx, sy, sz, sd = carry
        sl = pl.ds(ci * _CW, _CW)
        bits = bits_scratch[:, sl]
        maskb = bits <= t
        maskf = maskb.astype(jnp.float32)
        d2m = lax.bitcast_convert_type(bits, jnp.float32)
        dist = jnp.where(d2m > 1e-12, jnp.sqrt(d2m), 0.0)
        sx = sx + jnp.sum(maskf * coord_t_ref[0:1, sl], axis=1, keepdims=True)
        sy = sy + jnp.sum(maskf * coord_t_ref[1:2, sl], axis=1, keepdims=True)
        sz = sz + jnp.sum(maskf * coord_t_ref[2:3, sl], axis=1, keepdims=True)
        sd = sd + jnp.sum(jnp.where(maskb, dist, 0.0), axis=1, keepdims=True)
        return sx, sy, sz, sd

    zero = jnp.zeros((br, 1), jnp.float32)
    sx, sy, sz, sd = lax.fori_loop(c0, c1 + 1, mean_body,
                                   (zero, zero, zero, zero))
    mx = sx * inv_m
    my = sy * inv_m
    mz = sz * inv_m

    def cov_body(ci, carry):
        sxx, syy, szz, sxy, sxz, syz = carry
        sl = pl.ds(ci * _CW, _CW)
        maskf = (bits_scratch[:, sl] <= t).astype(jnp.float32)
        cx = (coord_t_ref[0:1, sl] - mx).astype(bf).astype(jnp.float32)
        cy = (coord_t_ref[1:2, sl] - my).astype(bf).astype(jnp.float32)
        cz = (coord_t_ref[2:3, sl] - mz).astype(bf).astype(jnp.float32)
        sxx = sxx + jnp.sum(maskf * (cx * cx), axis=1, keepdims=True)
        syy = syy + jnp.sum(maskf * (cy * cy), axis=1, keepdims=True)
        szz = szz + jnp.sum(maskf * (cz * cz), axis=1, keepdims=True)
        sxy = sxy + jnp.sum(maskf * (cx * cy), axis=1, keepdims=True)
        sxz = sxz + jnp.sum(maskf * (cx * cz), axis=1, keepdims=True)
        syz = syz + jnp.sum(maskf * (cy * cz), axis=1, keepdims=True)
        return sxx, syy, szz, sxy, sxz, syz

    sxx, syy, szz, sxy, sxz, syz = lax.fori_loop(
        c0, c1 + 1, cov_body, (zero, zero, zero, zero, zero, zero))
    r_km1 = jnp.float32(1.0 / (_K - 1))
    cxx = sxx * r_km1
    cyy = syy * r_km1
    czz = szz * r_km1
    cxy = sxy * r_km1
    cxz = sxz * r_km1
    cyz = syz * r_km1

    tr = cxx + cyy + czz
    pm = (cxx * cyy - cxy * cxy) + (cxx * czz - cxz * cxz) + (cyy * czz - cyz * cyz)
    det = (cxx * (cyy * czz - cyz * cyz)
           - cxy * (cxy * czz - cyz * cxz)
           + cxz * (cxy * cyz - cyy * cxz))

    q = tr * (1.0 / 3.0)
    p2 = ((cxx - q) ** 2 + (cyy - q) ** 2 + (czz - q) ** 2
          + 2.0 * (cxy * cxy + cxz * cxz + cyz * cyz))
    p = jnp.sqrt(jnp.maximum(p2, 0.0) * (1.0 / 6.0))

    def newton_body(_, x):
        f = ((x - tr) * x + pm) * x - det
        fp = (3.0 * x - 2.0 * tr) * x + pm
        delta = jnp.where(jnp.abs(fp) > 1e-30, f / fp, 0.0)
        return jnp.maximum(x - delta, q)

    lam = lax.fori_loop(0, 12, newton_body, q + 2.0 * p)

    lin = (2.0 * lam - tr) / tr
    den = 1.0 / (sd * inv_m + 1e-6)

    ok = cnt >= _K
    lin_ref[:, :] = jnp.where(ok, lin, 0.0)
    den_ref[:, :] = jnp.where(ok, den, 0.0)


def _mlp_kernel(feat_ref, w1t_ref, b1_ref, gamma_ref, beta_ref,
                w2t_ref, b2_ref, lin_ref, den_ref, out_ref):
    bf = jnp.bfloat16
    h = jnp.dot(feat_ref[:, :].astype(bf), w1t_ref[:, :].astype(bf),
                preferred_element_type=jnp.float32) + b1_ref[0:1, :]
    mu = jnp.mean(h, axis=0, keepdims=True)
    var = jnp.mean((h - mu) ** 2, axis=0, keepdims=True)
    hn = (h - mu) / jnp.sqrt(var + 1e-5) * gamma_ref[0:1, :] + beta_ref[0:1, :]
    hn = jnp.maximum(hn, 0.0)
    logits = jnp.dot(hn.astype(bf), w2t_ref[:, :].astype(bf),
                     preferred_element_type=jnp.float32) + b2_ref[0:1, :]

    mx = jnp.max(logits, axis=1, keepdims=True)
    e = jnp.exp(logits - mx)
    probs = e / jnp.sum(e, axis=1, keepdims=True)
    p0 = probs[:, 0:1]
    p1 = probs[:, 1:2]
    p2 = probs[:, 2:3]

    lin = lin_ref[:, :]
    den = den_ref[:, :]
    third = jnp.float32(1.0 / 3.0)
    tower = (den * 2.0 + p0) * third
    background = (jnp.maximum(1.0 - lin, 1.0 - den) + p1) * third
    line = (lin * 2.0 + p2) * third

    c01 = tower * 0.1 + background * 0.5 + line * 0.2 + 1e-6
    c2 = tower * 0.1 + background * 0.5 + line * 5.0 + 1e-6
    out_ref[:, :] = jnp.concatenate([c01, c01, c2], axis=1)


def kernel(feat, coord, batch, W1, b1, gamma, beta, W2, b2):
    n, _ = feat.shape
    batch = batch.astype(jnp.int32)
    coord = coord.astype(jnp.float32)

    block_rows = 128
    grid = n // block_rows

    coord_t = coord.T
    batch_row = batch.reshape(n, 1)
    batch_col = batch.reshape(1, n)

    b_first = batch[::block_rows]
    b_last = batch[block_rows - 1::block_rows]
    col_lo = jnp.searchsorted(batch, b_first, side="left").astype(jnp.int32)
    col_hi = jnp.searchsorted(batch, b_last, side="right").astype(jnp.int32)
    chunk_lo = col_lo // _CW
    chunk_hi = (jnp.maximum(col_hi, 1) - 1) // _CW
    win = jnp.stack([chunk_lo, chunk_hi], axis=1).reshape(-1)

    lin, den = pl.pallas_call(
        functools.partial(_geom_kernel, block_rows=block_rows),
        grid_spec=pltpu.PrefetchScalarGridSpec(
            num_scalar_prefetch=1,
            grid=(grid,),
            in_specs=[
                pl.BlockSpec((block_rows, 3), lambda i, w: (i, 0)),
                pl.BlockSpec((3, n), lambda i, w: (0, 0)),
                pl.BlockSpec((block_rows, 1), lambda i, w: (i, 0)),
                pl.BlockSpec((1, n), lambda i, w: (0, 0)),
            ],
            out_specs=[
                pl.BlockSpec((block_rows, 1), lambda i, w: (i, 0)),
                pl.BlockSpec((block_rows, 1), lambda i, w: (i, 0)),
            ],
            scratch_shapes=[pltpu.VMEM((block_rows, n), jnp.int32)],
        ),
        out_shape=[
            jax.ShapeDtypeStruct((n, 1), jnp.float32),
            jax.ShapeDtypeStruct((n, 1), jnp.float32),
        ],
    )(win, coord, coord_t, batch_row, batch_col)

    out = pl.pallas_call(
        _mlp_kernel,
        out_shape=jax.ShapeDtypeStruct((n, 3), jnp.float32),
    )(feat.astype(jnp.float32), W1.T.astype(jnp.float32),
      b1.reshape(1, -1).astype(jnp.float32),
      gamma.reshape(1, -1).astype(jnp.float32),
      beta.reshape(1, -1).astype(jnp.float32),
      W2.T.astype(jnp.float32), b2.reshape(1, -1).astype(jnp.float32),
      lin, den)
    return out
